# baseline (speedup 1.0000x reference)
import functools

import jax
import jax.numpy as jnp
from jax import lax
from jax.experimental import pallas as pl
from jax.experimental.pallas import tpu as pltpu
from jax.experimental.pallas import tpu_sc as plsc

COORD_DIM = 3
NUM_FREQ = 32
MAX_POS = 10000.0
RADIUS = 0.08
EMB = 2 * NUM_FREQ * COORD_DIM

NQ = 4096
NY = 10000
NYPAD = 10240
NWORD = 640
D = 128

NSUB = 32
QPER = NQ // NSUB
ECAP = 4096
ETOT = NSUB * ECAP
GCH = 256
EB = 256
EBLKS = ECAP // EB


def _emb_selector():
    freqs = (1.0 / MAX_POS) ** (jnp.arange(NUM_FREQ, dtype=jnp.float32) / NUM_FREQ)
    col = jnp.arange(EMB)
    c = col // (2 * NUM_FREQ)
    f = (col % (2 * NUM_FREQ)) // 2
    S = (jnp.arange(COORD_DIM)[:, None] == c[None, :]).astype(jnp.float32)
    return S * freqs[f][None, :]


def _sinusoid(P):
    par = lax.broadcasted_iota(jnp.int32, P.shape, 1) % 2
    return jnp.where(par == 0, jnp.sin(P), jnp.cos(P))


def _phases(pb, s_ref):
    return (pb[:, 0:1] * s_ref[0:1, :]
            + pb[:, 1:2] * s_ref[1:2, :]
            + pb[:, 2:3] * s_ref[2:3, :])


def _stage_a_body(x_ref, ypb_ref, yp_ref, s_ref, wy_ref, wx_ref, b_ref,
                  ax_ref, ay_ref, bits_ref):
    emb_y = _sinusoid(_phases(ypb_ref[...], s_ref))
    ay_ref[...] = jnp.dot(emb_y, wy_ref[...], preferred_element_type=jnp.float32)
    xb = x_ref[...]
    emb_x = _sinusoid(_phases(xb, s_ref))
    ax_ref[...] = (jnp.dot(emb_x, wx_ref[...], preferred_element_type=jnp.float32)
                   + b_ref[...])
    yp = yp_ref[...]
    xsq = jnp.sum(xb * xb, axis=1, keepdims=True)
    ysq = jnp.sum(yp * yp, axis=1)[None, :]
    cross = lax.dot_general(xb, yp, (((1,), (1,)), ((), ())),
                            preferred_element_type=jnp.float32)
    sq = xsq + ysq - 2.0 * cross
    m = (sq <= RADIUS * RADIUS).astype(jnp.float32)
    acc = jnp.zeros((xb.shape[0], NWORD), jnp.float32)
    for k in range(16):
        acc = acc + m[:, k * NWORD:(k + 1) * NWORD] * float(1 << k)
    bits_ref[...] = acc.astype(jnp.int32)


def _stage_a(x, ypad, S, W0y, W0x, b0):
    xblk = NQ // 8
    yblk = NYPAD // 8
    return pl.pallas_call(
        _stage_a_body,
        grid=(8,),
        in_specs=[
            pl.BlockSpec((xblk, COORD_DIM), lambda i: (i, 0)),
            pl.BlockSpec((yblk, COORD_DIM), lambda i: (i, 0)),
            pl.BlockSpec((NYPAD, COORD_DIM), lambda i: (0, 0)),
            pl.BlockSpec((COORD_DIM, EMB), lambda i: (0, 0)),
            pl.BlockSpec((EMB, D), lambda i: (0, 0)),
            pl.BlockSpec((EMB, D), lambda i: (0, 0)),
            pl.BlockSpec((1, D), lambda i: (0, 0)),
        ],
        out_specs=[
            pl.BlockSpec((xblk, D), lambda i: (i, 0)),
            pl.BlockSpec((yblk, D), lambda i: (i, 0)),
            pl.BlockSpec((xblk, NWORD), lambda i: (i, 0)),
        ],
        out_shape=[
            jax.ShapeDtypeStruct((NQ, D), jnp.float32),
            jax.ShapeDtypeStruct((NYPAD, D), jnp.float32),
            jax.ShapeDtypeStruct((NQ, NWORD), jnp.int32),
        ],
    )(x, ypad, ypad, S, W0y, W0x, b0.reshape(1, D))


def _sc_compact_gather(bits, ay, fy):
    mesh = plsc.VectorSubcoreMesh(core_axis_name="c", subcore_axis_name="s")

    @functools.partial(
        pl.kernel,
        out_type=(
            jax.ShapeDtypeStruct((ETOT, D), jnp.float32),
            jax.ShapeDtypeStruct((ETOT, D), jnp.float32),
            jax.ShapeDtypeStruct((ETOT,), jnp.int32),
            jax.ShapeDtypeStruct((NSUB, 16), jnp.int32),
        ),
        mesh=mesh,
        compiler_params=pltpu.CompilerParams(needs_layout_passes=False),
        scratch_types=[
            pltpu.VMEM((32 * NWORD + 16,), jnp.int32),
            pltpu.VMEM((ECAP + 128,), jnp.int32),
            pltpu.VMEM((ECAP + 128,), jnp.int32),
            pltpu.VMEM((NWORD + 32,), jnp.int32),
            pltpu.VMEM((GCH, D), jnp.float32),
            pltpu.VMEM((GCH, D), jnp.float32),
            pltpu.VMEM((16,), jnp.int32),
            pltpu.SemaphoreType.DMA,
            pltpu.SemaphoreType.DMA,
        ],
    )
    def body(bits_hbm, ay_hbm, fy_hbm, gay_hbm, gfy_hbm, dst_hbm, cnt_hbm,
             bits_v, src_v, dst_v, hitw_v, rows_a, rows_b, cnt_v, sem_a, sem_b):
        wid = lax.axis_index("s") * 2 + lax.axis_index("c")
        q0 = wid * QPER
        lane = lax.iota(jnp.int32, 16)
        zeros16 = jnp.zeros((16,), jnp.int32)

        def zbody(i, carry):
            src_v[pl.ds(i * 16, 16)] = zeros16
            dst_v[pl.ds(i * 16, 16)] = zeros16
            return carry
        lax.fori_loop(0, (ECAP + 128) // 16, zbody, 0)

        def qchunk(qc, ptr):
            pltpu.sync_copy(bits_hbm.at[pl.ds((q0 + qc * 32) * NWORD, 32 * NWORD)],
                            bits_v.at[pl.ds(0, 32 * NWORD)])

            def per_q(li, ptr):
                base = li * NWORD

                def pa(wg, p):
                    bw = bits_v[pl.ds(base + wg * 16, 16)]
                    mnz = bw != 0
                    plsc.store_compressed(hitw_v.at[pl.ds(p, 16)],
                                          wg * 16 + lane, mask=mnz)
                    return p + plsc.all_reduce_population_count(mnz)[0]
                nhit = lax.fori_loop(0, NWORD // 16, pa, 0)
                dloc = qc * 32 + li

                def pb(t, ptr):
                    w_idx = hitw_v[pl.ds(t, 16)][0]
                    w = bits_v[pl.ds(base + w_idx, 16)][0]
                    m16 = ((w >> lane) & 1) == 1
                    jv = w_idx + NWORD * lane
                    sp = jnp.minimum(ptr, ECAP)
                    plsc.store_compressed(src_v.at[pl.ds(sp, 16)], jv, mask=m16)
                    plsc.store_compressed(dst_v.at[pl.ds(sp, 16)],
                                          jnp.broadcast_to(dloc, (16,)).astype(jnp.int32),
                                          mask=m16)
                    return ptr + plsc.all_reduce_population_count(m16)[0]
                return lax.fori_loop(0, nhit, pb, ptr)
            return lax.fori_loop(0, 32, per_q, ptr)

        nedge = lax.fori_loop(0, QPER // 32, qchunk, 0)
        nedge = jnp.minimum(nedge, ECAP)

        def chunk(c, carry):
            idx = src_v.at[pl.ds(c * GCH, GCH)]
            off = wid * ECAP + c * GCH
            ga = pltpu.async_copy(ay_hbm.at[idx], rows_a, sem_a)
            gb = pltpu.async_copy(fy_hbm.at[idx], rows_b, sem_b)
            ga.wait()
            pltpu.sync_copy(rows_a, gay_hbm.at[pl.ds(off, GCH)])
            gb.wait()
            pltpu.sync_copy(rows_b, gfy_hbm.at[pl.ds(off, GCH)])
            pltpu.sync_copy(dst_v.at[pl.ds(c * GCH, GCH)],
                            dst_hbm.at[pl.ds(off, GCH)])
            return carry
        nch = (nedge + GCH - 1) // GCH
        lax.fori_loop(0, nch, chunk, 0)

        cnt_v[pl.ds(0, 16)] = jnp.broadcast_to(nedge, (16,)).astype(jnp.int32)
        pltpu.sync_copy(cnt_v, cnt_hbm.at[wid])

    return body(bits, ay, fy)


def _gelu(h):
    return 0.5 * h * (1.0 + lax.erf(h * 0.7071067811865476))


def _mlp_body(cnt_ref, gay_ref, gfy_ref, dst_ref, ax_ref,
              w1_ref, b1_ref, w2_ref, b2_ref, w3_ref, b3_ref, out_ref):
    s = pl.program_id(0)
    e = pl.program_id(1)
    count = cnt_ref[s]

    @pl.when(e == 0)
    def _():
        out_ref[...] = jnp.zeros_like(out_ref)

    @pl.when(e * EB < count)
    def _():
        dst = dst_ref[...].reshape(1, EB)
        ecol = lax.broadcasted_iota(jnp.int32, (1, EB), 1) + e * EB
        vcol = ecol < count
        dstm = jnp.where(vcol, dst, -1)
        qrow = lax.broadcasted_iota(jnp.int32, (QPER, EB), 0)
        oh = (qrow == dstm).astype(jnp.float32)

        gx = lax.dot_general(oh, ax_ref[...], (((0,), (0,)), ((), ())),
                             preferred_element_type=jnp.float32)
        h = _gelu(gay_ref[...] + gx)
        h = _gelu(jnp.dot(h, w1_ref[...], preferred_element_type=jnp.float32)
                  + b1_ref[...])
        h = _gelu(jnp.dot(h, w2_ref[...], preferred_element_type=jnp.float32)
                  + b2_ref[...])
        k = jnp.dot(h, w3_ref[...], preferred_element_type=jnp.float32) + b3_ref[...]
        k = k * gfy_ref[...]
        vrow = (lax.broadcasted_iota(jnp.int32, (EB, 1), 0) + e * EB) < count
        k = jnp.where(vrow, k, 0.0)
        out_ref[...] += lax.dot_general(oh, k, (((1,), (0,)), ((), ())),
                                        preferred_element_type=jnp.float32)


def _mlp_scatter(counts, gay, gfy, dst3, ax, W1, b1, W2, b2, W3, b3):
    grid_spec = pltpu.PrefetchScalarGridSpec(
        num_scalar_prefetch=1,
        grid=(NSUB, EBLKS),
        in_specs=[
            pl.BlockSpec((EB, D), lambda s, e, c: (s * EBLKS + e, 0)),
            pl.BlockSpec((EB, D), lambda s, e, c: (s * EBLKS + e, 0)),
            pl.BlockSpec((1, 1, EB), lambda s, e, c: (s * EBLKS + e, 0, 0)),
            pl.BlockSpec((QPER, D), lambda s, e, c: (s, 0)),
            pl.BlockSpec((D, 256), lambda s, e, c: (0, 0)),
            pl.BlockSpec((1, 256), lambda s, e, c: (0, 0)),
            pl.BlockSpec((256, D), lambda s, e, c: (0, 0)),
            pl.BlockSpec((1, D), lambda s, e, c: (0, 0)),
            pl.BlockSpec((D, D), lambda s, e, c: (0, 0)),
            pl.BlockSpec((1, D), lambda s, e, c: (0, 0)),
        ],
        out_specs=pl.BlockSpec((QPER, D), lambda s, e, c: (s, 0)),
    )
    return pl.pallas_call(
        _mlp_body,
        grid_spec=grid_spec,
        out_shape=jax.ShapeDtypeStruct((NQ, D), jnp.float32),
        compiler_params=pltpu.CompilerParams(
            dimension_semantics=("parallel", "arbitrary")),
    )(counts, gay, gfy, dst3, ax,
      W1, b1.reshape(1, 256), W2, b2.reshape(1, D), W3, b3.reshape(1, D))


def kernel(y, x, f_y, W0, b0, W1, b1, W2, b2, W3, b3):
    S = _emb_selector()
    W0y = W0[:EMB]
    W0x = W0[EMB:]

    ypad = jnp.concatenate(
        [y, jnp.full((NYPAD - NY, COORD_DIM), 100.0, jnp.float32)], axis=0)
    ax, ay, bits = _stage_a(x, ypad, S, W0y, W0x, b0)

    gay, gfy, dst, counts = _sc_compact_gather(bits.reshape(-1), ay, f_y)

    dst3 = dst.reshape(ETOT // EB, 1, EB)
    return _mlp_scatter(counts[:, 0], gay, gfy, dst3, ax,
                        W1, b1, W2, b2, W3, b3)


# speedup vs baseline: 1.2328x; 1.2328x over previous
import functools

import jax
import jax.numpy as jnp
from jax import lax
from jax.experimental import pallas as pl
from jax.experimental.pallas import tpu as pltpu
from jax.experimental.pallas import tpu_sc as plsc

COORD_DIM = 3
NUM_FREQ = 32
MAX_POS = 10000.0
RADIUS = 0.08
EMB = 2 * NUM_FREQ * COORD_DIM

NQ = 4096
NY = 10000
NYPAD = 10240
NWORD = 640
D = 128

NSUB = 32
QPER = NQ // NSUB
ECAP = 4096
ETOT = NSUB * ECAP
GCH = 256
EB = 512
EBLKS = ECAP // EB


def _emb_selector():
    freqs = (1.0 / MAX_POS) ** (jnp.arange(NUM_FREQ, dtype=jnp.float32) / NUM_FREQ)
    col = jnp.arange(EMB)
    c = col // (2 * NUM_FREQ)
    f = (col % (2 * NUM_FREQ)) // 2
    S = (jnp.arange(COORD_DIM)[:, None] == c[None, :]).astype(jnp.float32)
    return S * freqs[f][None, :]


def _sinusoid(P):
    par = lax.broadcasted_iota(jnp.int32, P.shape, 1) % 2
    return jnp.where(par == 0, jnp.sin(P), jnp.cos(P))


def _phases(pb, s_ref):
    return (pb[:, 0:1] * s_ref[0:1, :]
            + pb[:, 1:2] * s_ref[1:2, :]
            + pb[:, 2:3] * s_ref[2:3, :])


def _stage_a_body(x_ref, ypb_ref, yp_ref, s_ref, wy_ref, wx_ref, b_ref,
                  ax_ref, ay_ref, bits_ref):
    emb_y = _sinusoid(_phases(ypb_ref[...], s_ref))
    ay_ref[...] = jnp.dot(emb_y, wy_ref[...], preferred_element_type=jnp.float32)
    xb = x_ref[...]
    emb_x = _sinusoid(_phases(xb, s_ref))
    ax_ref[...] = (jnp.dot(emb_x, wx_ref[...], preferred_element_type=jnp.float32)
                   + b_ref[...])
    yp = yp_ref[...]
    xsq = jnp.sum(xb * xb, axis=1, keepdims=True)
    ysq = jnp.sum(yp * yp, axis=1)[None, :]
    cross = lax.dot_general(xb, yp, (((1,), (1,)), ((), ())),
                            preferred_element_type=jnp.float32)
    sq = xsq + ysq - 2.0 * cross
    m = (sq <= RADIUS * RADIUS).astype(jnp.float32)
    acc = jnp.zeros((xb.shape[0], NWORD), jnp.float32)
    for k in range(16):
        acc = acc + m[:, k * NWORD:(k + 1) * NWORD] * float(1 << k)
    bits_ref[...] = acc.astype(jnp.int32)


def _stage_a(x, ypad, S, W0y, W0x, b0):
    xblk = NQ // 8
    yblk = NYPAD // 8
    return pl.pallas_call(
        _stage_a_body,
        grid=(8,),
        in_specs=[
            pl.BlockSpec((xblk, COORD_DIM), lambda i: (i, 0)),
            pl.BlockSpec((yblk, COORD_DIM), lambda i: (i, 0)),
            pl.BlockSpec((NYPAD, COORD_DIM), lambda i: (0, 0)),
            pl.BlockSpec((COORD_DIM, EMB), lambda i: (0, 0)),
            pl.BlockSpec((EMB, D), lambda i: (0, 0)),
            pl.BlockSpec((EMB, D), lambda i: (0, 0)),
            pl.BlockSpec((1, D), lambda i: (0, 0)),
        ],
        out_specs=[
            pl.BlockSpec((xblk, D), lambda i: (i, 0)),
            pl.BlockSpec((yblk, D), lambda i: (i, 0)),
            pl.BlockSpec((xblk, NWORD), lambda i: (i, 0)),
        ],
        out_shape=[
            jax.ShapeDtypeStruct((NQ, D), jnp.float32),
            jax.ShapeDtypeStruct((NYPAD, D), jnp.float32),
            jax.ShapeDtypeStruct((NQ, NWORD), jnp.int32),
        ],
    )(x, ypad, ypad, S, W0y, W0x, b0.reshape(1, D))


def _sc_compact_gather(bits, ay, fy):
    mesh = plsc.VectorSubcoreMesh(core_axis_name="c", subcore_axis_name="s")

    @functools.partial(
        pl.kernel,
        out_type=(
            jax.ShapeDtypeStruct((ETOT, D), jnp.float32),
            jax.ShapeDtypeStruct((ETOT, D), jnp.float32),
            jax.ShapeDtypeStruct((ETOT,), jnp.int32),
            jax.ShapeDtypeStruct((NSUB, 16), jnp.int32),
        ),
        mesh=mesh,
        compiler_params=pltpu.CompilerParams(needs_layout_passes=False),
        scratch_types=[
            pltpu.VMEM((32 * NWORD + 16,), jnp.int32),
            pltpu.VMEM((ECAP + 128,), jnp.int32),
            pltpu.VMEM((ECAP + 128,), jnp.int32),
            pltpu.VMEM((NWORD + 32,), jnp.int32),
            pltpu.VMEM((GCH, D), jnp.float32),
            pltpu.VMEM((GCH, D), jnp.float32),
            pltpu.VMEM((16,), jnp.int32),
            pltpu.SemaphoreType.DMA,
            pltpu.SemaphoreType.DMA,
        ],
    )
    def body(bits_hbm, ay_hbm, fy_hbm, gay_hbm, gfy_hbm, dst_hbm, cnt_hbm,
             bits_v, src_v, dst_v, hitw_v, rows_a, rows_b, cnt_v, sem_a, sem_b):
        wid = lax.axis_index("s") * 2 + lax.axis_index("c")
        q0 = wid * QPER
        lane = lax.iota(jnp.int32, 16)
        zeros16 = jnp.zeros((16,), jnp.int32)

        def zbody(i, carry):
            src_v[pl.ds(i * 16, 16)] = zeros16
            dst_v[pl.ds(i * 16, 16)] = zeros16
            return carry
        lax.fori_loop(0, (ECAP + 128) // 16, zbody, 0)

        def qchunk(qc, ptr):
            pltpu.sync_copy(bits_hbm.at[pl.ds((q0 + qc * 32) * NWORD, 32 * NWORD)],
                            bits_v.at[pl.ds(0, 32 * NWORD)])

            def per_q(li, ptr):
                base = li * NWORD

                def pa(wg, p):
                    bw = bits_v[pl.ds(base + wg * 16, 16)]
                    mnz = bw != 0
                    plsc.store_compressed(hitw_v.at[pl.ds(p, 16)],
                                          wg * 16 + lane, mask=mnz)
                    return p + plsc.all_reduce_population_count(mnz)[0]
                nhit = lax.fori_loop(0, NWORD // 16, pa, 0)
                dloc = qc * 32 + li

                def pb(t, ptr):
                    w_idx = hitw_v[pl.ds(t, 16)][0]
                    w = bits_v[pl.ds(base + w_idx, 16)][0]
                    m16 = ((w >> lane) & 1) == 1
                    jv = w_idx + NWORD * lane
                    sp = jnp.minimum(ptr, ECAP)
                    plsc.store_compressed(src_v.at[pl.ds(sp, 16)], jv, mask=m16)
                    plsc.store_compressed(dst_v.at[pl.ds(sp, 16)],
                                          jnp.broadcast_to(dloc, (16,)).astype(jnp.int32),
                                          mask=m16)
                    return ptr + plsc.all_reduce_population_count(m16)[0]
                return lax.fori_loop(0, nhit, pb, ptr)
            return lax.fori_loop(0, 32, per_q, ptr)

        nedge = lax.fori_loop(0, QPER // 32, qchunk, 0)
        nedge = jnp.minimum(nedge, ECAP)

        def chunk(c, carry):
            idx = src_v.at[pl.ds(c * GCH, GCH)]
            off = wid * ECAP + c * GCH
            ga = pltpu.async_copy(ay_hbm.at[idx], rows_a, sem_a)
            gb = pltpu.async_copy(fy_hbm.at[idx], rows_b, sem_b)
            ga.wait()
            pltpu.sync_copy(rows_a, gay_hbm.at[pl.ds(off, GCH)])
            gb.wait()
            pltpu.sync_copy(rows_b, gfy_hbm.at[pl.ds(off, GCH)])
            pltpu.sync_copy(dst_v.at[pl.ds(c * GCH, GCH)],
                            dst_hbm.at[pl.ds(off, GCH)])
            return carry
        nch = (nedge + GCH - 1) // GCH
        lax.fori_loop(0, nch, chunk, 0)

        cnt_v[pl.ds(0, 16)] = jnp.broadcast_to(nedge, (16,)).astype(jnp.int32)
        pltpu.sync_copy(cnt_v, cnt_hbm.at[wid])

    return body(bits, ay, fy)


def _gelu(h):
    return 0.5 * h * (1.0 + lax.erf(h * 0.7071067811865476))


def _mlp_body(cnt_ref, gay_ref, gfy_ref, dst_ref, ax_ref,
              w1_ref, b1_ref, w2_ref, b2_ref, w3_ref, b3_ref, out_ref):
    s = pl.program_id(0)
    e = pl.program_id(1)
    count = cnt_ref[s]

    @pl.when(e == 0)
    def _():
        out_ref[...] = jnp.zeros_like(out_ref)

    @pl.when(e * EB < count)
    def _():
        dst = dst_ref[...].reshape(1, EB)
        ecol = lax.broadcasted_iota(jnp.int32, (1, EB), 1) + e * EB
        vcol = ecol < count
        dstm = jnp.where(vcol, dst, -1)
        qrow = lax.broadcasted_iota(jnp.int32, (QPER, EB), 0)
        oh = (qrow == dstm).astype(jnp.float32)

        gx = lax.dot_general(oh, ax_ref[...], (((0,), (0,)), ((), ())),
                             preferred_element_type=jnp.float32)
        h = _gelu(gay_ref[...] + gx)
        h = _gelu(jnp.dot(h, w1_ref[...], preferred_element_type=jnp.float32)
                  + b1_ref[...])
        h = _gelu(jnp.dot(h, w2_ref[...], preferred_element_type=jnp.float32)
                  + b2_ref[...])
        k = jnp.dot(h, w3_ref[...], preferred_element_type=jnp.float32) + b3_ref[...]
        k = k * gfy_ref[...]
        vrow = (lax.broadcasted_iota(jnp.int32, (EB, 1), 0) + e * EB) < count
        k = jnp.where(vrow, k, 0.0)
        out_ref[...] += lax.dot_general(oh, k, (((1,), (0,)), ((), ())),
                                        preferred_element_type=jnp.float32)


def _mlp_scatter(counts, gay, gfy, dst3, ax, W1, b1, W2, b2, W3, b3):
    grid_spec = pltpu.PrefetchScalarGridSpec(
        num_scalar_prefetch=1,
        grid=(NSUB, EBLKS),
        in_specs=[
            pl.BlockSpec((EB, D), lambda s, e, c: (s * EBLKS + e, 0)),
            pl.BlockSpec((EB, D), lambda s, e, c: (s * EBLKS + e, 0)),
            pl.BlockSpec((1, 1, EB), lambda s, e, c: (s * EBLKS + e, 0, 0)),
            pl.BlockSpec((QPER, D), lambda s, e, c: (s, 0)),
            pl.BlockSpec((D, 256), lambda s, e, c: (0, 0)),
            pl.BlockSpec((1, 256), lambda s, e, c: (0, 0)),
            pl.BlockSpec((256, D), lambda s, e, c: (0, 0)),
            pl.BlockSpec((1, D), lambda s, e, c: (0, 0)),
            pl.BlockSpec((D, D), lambda s, e, c: (0, 0)),
            pl.BlockSpec((1, D), lambda s, e, c: (0, 0)),
        ],
        out_specs=pl.BlockSpec((QPER, D), lambda s, e, c: (s, 0)),
    )
    return pl.pallas_call(
        _mlp_body,
        grid_spec=grid_spec,
        out_shape=jax.ShapeDtypeStruct((NQ, D), jnp.float32),
        compiler_params=pltpu.CompilerParams(
            dimension_semantics=("parallel", "arbitrary")),
    )(counts, gay, gfy, dst3, ax,
      W1, b1.reshape(1, 256), W2, b2.reshape(1, D), W3, b3.reshape(1, D))


def kernel(y, x, f_y, W0, b0, W1, b1, W2, b2, W3, b3):
    S = _emb_selector()
    W0y = W0[:EMB]
    W0x = W0[EMB:]

    ypad = jnp.concatenate(
        [y, jnp.full((NYPAD - NY, COORD_DIM), 100.0, jnp.float32)], axis=0)
    ax, ay, bits = _stage_a(x, ypad, S, W0y, W0x, b0)

    gay, gfy, dst, counts = _sc_compact_gather(bits.reshape(-1), ay, f_y)

    dst3 = dst.reshape(ETOT // EB, 1, EB)
    return _mlp_scatter(counts[:, 0], gay, gfy, dst3, ax,
                        W1, b1, W2, b2, W3, b3)
